# baseline (speedup 1.0000x reference)
import jax
import jax.numpy as jnp
from jax.experimental import pallas as pl
from jax.experimental.pallas import tpu as pltpu

E = 8
D = 1024
H_DIM = 2048
MAX_A = 32
T = 8
B = 128
CB = 16
NC = 32
DC = D // NC

_INV_SQRT2 = 0.7071067811865476


def _mlp_kernel(meta_hbm, x_ref, w1_hbm, b1_ref, w2_ref,
                out_ref, xs_ref, w1_buf, meta_s, sems, msem):
    e = pl.program_id(0)
    slot = jax.lax.rem(e, 2)
    nslot = jax.lax.rem(e + 1, 2)

    @pl.when(e == 0)
    def _():
        pltpu.make_async_copy(meta_hbm, meta_s, msem).start()
        for c in range(NC):
            pltpu.make_async_copy(
                w1_hbm.at[0, pl.ds(c * DC, DC), :],
                w1_buf.at[0, pl.ds(c * DC, DC), :],
                sems.at[0, c]).start()
        pltpu.make_async_copy(meta_hbm, meta_s, msem).wait()

    @pl.when(e + 1 < E)
    def _():
        for c in range(NC):
            pltpu.make_async_copy(
                w1_hbm.at[e + 1, pl.ds(c * DC, DC), :],
                w1_buf.at[nslot, pl.ds(c * DC, DC), :],
                sems.at[nslot, c]).start()

    for c in range(NC):
        pltpu.make_async_copy(
            w1_hbm.at[e, pl.ds(c * DC, DC), :],
            w1_buf.at[slot, pl.ds(c * DC, DC), :],
            sems.at[slot, c]).wait()

    cnt = meta_s[E * B + e]
    nb = (cnt + CB - 1) // CB

    def blk(k, carry):
        base = k * CB
        for i in range(CB):
            b = meta_s[e * B + base + i]
            xs_ref[pl.ds(i * T, T), :] = x_ref[pl.ds(b * T, T), :]
        h = jnp.dot(xs_ref[...], w1_buf[slot],
                    preferred_element_type=jnp.float32) + b1_ref[0, 0]
        h = 0.5 * h * (1.0 + jax.lax.erf(h * _INV_SQRT2))
        dec = (jnp.dot(h, w2_ref[0], preferred_element_type=jnp.float32)
               + b1_ref[0, 1, :MAX_A])
        for i in range(CB):
            b = meta_s[e * B + base + i]

            @pl.when(base + i < cnt)
            def _store():
                out_ref[pl.ds(b * T, T), :] = dec[i * T:(i + 1) * T, :]

        return carry

    jax.lax.fori_loop(0, nb, blk, 0)


def kernel(pred_action_latents, embodiment_ids, W1, b1, W2, b2, action_mask):
    Bn, Tn, _ = pred_action_latents.shape
    N = Bn * Tn
    x = pred_action_latents.reshape(N, D)

    W2m = W2 * action_mask[:, None, :]
    b1m = jnp.concatenate(
        [b1[:, None, :],
         jnp.pad(b2 * action_mask, ((0, 0), (0, H_DIM - MAX_A)))[:, None, :]],
        axis=1)

    ids = embodiment_ids.astype(jnp.int32)
    order = jnp.argsort(ids, stable=True).astype(jnp.int32)
    sorted_ids = ids[order]
    counts = jnp.zeros((E,), jnp.int32).at[ids].add(1)
    starts = jnp.concatenate(
        [jnp.zeros((1,), jnp.int32), jnp.cumsum(counts)[:-1].astype(jnp.int32)])
    local = jnp.arange(Bn, dtype=jnp.int32) - starts[sorted_ids]
    ebidx = jnp.zeros((E * Bn,), jnp.int32).at[sorted_ids * Bn + local].set(order)
    meta = jnp.concatenate([ebidx, counts])

    out = pl.pallas_call(
        _mlp_kernel,
        grid=(E,),
        in_specs=[
            pl.BlockSpec(memory_space=pl.ANY),
            pl.BlockSpec((N, D), lambda e: (0, 0)),
            pl.BlockSpec(memory_space=pl.ANY),
            pl.BlockSpec((1, 2, H_DIM), lambda e: (e, 0, 0)),
            pl.BlockSpec((1, H_DIM, MAX_A), lambda e: (e, 0, 0)),
        ],
        out_specs=pl.BlockSpec((N, MAX_A), lambda e: (0, 0)),
        scratch_shapes=[
            pltpu.VMEM((CB * T, D), jnp.float32),
            pltpu.VMEM((2, D, H_DIM), jnp.float32),
            pltpu.SMEM((E * B + E,), jnp.int32),
            pltpu.SemaphoreType.DMA((2, NC)),
            pltpu.SemaphoreType.DMA,
        ],
        out_shape=jax.ShapeDtypeStruct((N, MAX_A), jnp.float32),
    )(meta, x, W1, b1m, W2m)
    return out.reshape(Bn, Tn, MAX_A)


# speedup vs baseline: 1.1529x; 1.1529x over previous
import jax
import jax.numpy as jnp
from jax.experimental import pallas as pl
from jax.experimental.pallas import tpu as pltpu

E = 8
D = 1024
H_DIM = 2048
MAX_A = 32
T = 8
B = 128
CB = 16
NC = 32
DC = D // NC

_INV_SQRT2 = 0.7071067811865476


def _mlp_kernel(meta_hbm, x_ref, w1_hbm, b1_ref, w2_ref,
                b2_ref, mask_ref, out_ref, xs_ref, w1_buf, meta_s, sems, msem):
    e = pl.program_id(0)
    slot = jax.lax.rem(e, 2)
    nslot = jax.lax.rem(e + 1, 2)

    @pl.when(e == 0)
    def _():
        pltpu.make_async_copy(meta_hbm, meta_s, msem).start()
        for c in range(NC):
            pltpu.make_async_copy(
                w1_hbm.at[0, pl.ds(c * DC, DC), :],
                w1_buf.at[0, pl.ds(c * DC, DC), :],
                sems.at[0, c]).start()
        pltpu.make_async_copy(meta_hbm, meta_s, msem).wait()

    @pl.when(e + 1 < E)
    def _():
        for c in range(NC):
            pltpu.make_async_copy(
                w1_hbm.at[e + 1, pl.ds(c * DC, DC), :],
                w1_buf.at[nslot, pl.ds(c * DC, DC), :],
                sems.at[nslot, c]).start()

    for c in range(NC):
        pltpu.make_async_copy(
            w1_hbm.at[e, pl.ds(c * DC, DC), :],
            w1_buf.at[slot, pl.ds(c * DC, DC), :],
            sems.at[slot, c]).wait()

    cnt = meta_s[E * B + e]
    nb = (cnt + CB - 1) // CB

    def blk(k, carry):
        base = k * CB
        for i in range(CB):
            b = meta_s[e * B + base + i]
            xs_ref[pl.ds(i * T, T), :] = x_ref[pl.ds(b * T, T), :]
        h = jnp.dot(xs_ref[...], w1_buf[slot],
                    preferred_element_type=jnp.float32) + b1_ref[0]
        h = 0.5 * h * (1.0 + jax.lax.erf(h * _INV_SQRT2))
        dec = jnp.dot(h, w2_ref[0], preferred_element_type=jnp.float32)
        dec = (dec + b2_ref[0]) * mask_ref[0]
        for i in range(CB):
            b = meta_s[e * B + base + i]

            @pl.when(base + i < cnt)
            def _store():
                out_ref[pl.ds(b * T, T), :] = dec[i * T:(i + 1) * T, :]

        return carry

    jax.lax.fori_loop(0, nb, blk, 0)


def kernel(pred_action_latents, embodiment_ids, W1, b1, W2, b2, action_mask):
    Bn, Tn, _ = pred_action_latents.shape
    N = Bn * Tn
    x = pred_action_latents.reshape(N, D)

    ids = embodiment_ids.astype(jnp.int32)
    order = jnp.argsort(ids, stable=True).astype(jnp.int32)
    sorted_ids = ids[order]
    counts = jnp.zeros((E,), jnp.int32).at[ids].add(1)
    starts = jnp.concatenate(
        [jnp.zeros((1,), jnp.int32), jnp.cumsum(counts)[:-1].astype(jnp.int32)])
    local = jnp.arange(Bn, dtype=jnp.int32) - starts[sorted_ids]
    ebidx = jnp.zeros((E * Bn,), jnp.int32).at[sorted_ids * Bn + local].set(order)
    meta = jnp.concatenate([ebidx, counts])

    out = pl.pallas_call(
        _mlp_kernel,
        grid=(E,),
        in_specs=[
            pl.BlockSpec(memory_space=pl.ANY),
            pl.BlockSpec((N, D), lambda e: (0, 0)),
            pl.BlockSpec(memory_space=pl.ANY),
            pl.BlockSpec((1, 1, H_DIM), lambda e: (e, 0, 0)),
            pl.BlockSpec((1, H_DIM, MAX_A), lambda e: (e, 0, 0)),
            pl.BlockSpec((1, 1, MAX_A), lambda e: (e, 0, 0)),
            pl.BlockSpec((1, 1, MAX_A), lambda e: (e, 0, 0)),
        ],
        out_specs=pl.BlockSpec((N, MAX_A), lambda e: (0, 0)),
        scratch_shapes=[
            pltpu.VMEM((CB * T, D), jnp.float32),
            pltpu.VMEM((2, D, H_DIM), jnp.float32),
            pltpu.SMEM((E * B + E,), jnp.int32),
            pltpu.SemaphoreType.DMA((2, NC)),
            pltpu.SemaphoreType.DMA,
        ],
        out_shape=jax.ShapeDtypeStruct((N, MAX_A), jnp.float32),
    )(meta, x, W1, b1[:, None, :], W2, b2[:, None, :],
      action_mask[:, None, :])
    return out.reshape(Bn, Tn, MAX_A)


# speedup vs baseline: 1.2714x; 1.1027x over previous
import jax
import jax.numpy as jnp
from jax.experimental import pallas as pl
from jax.experimental.pallas import tpu as pltpu

E = 8
D = 1024
H_DIM = 2048
MAX_A = 32
T = 8
B = 128
CB = 16
NC = 32
DC = D // NC
NW = B // 4
META_LEN = NW + 2 * E

_INV_SQRT2 = 0.7071067811865476


def _mlp_kernel(meta_hbm, x_ref, w1_hbm, b1_ref, w2_ref, b2_ref, mask_ref,
                out_ref, xs_ref, w1_buf, meta_s, sems, msem):
    e = pl.program_id(0)
    slot = jax.lax.rem(e, 2)
    nslot = jax.lax.rem(e + 1, 2)

    @pl.when(e == 0)
    def _():
        pltpu.make_async_copy(meta_hbm, meta_s, msem).start()
        for c in range(NC):
            pltpu.make_async_copy(
                w1_hbm.at[0, pl.ds(c * DC, DC), :],
                w1_buf.at[0, pl.ds(c * DC, DC), :],
                sems.at[0, c]).start()
        pltpu.make_async_copy(meta_hbm, meta_s, msem).wait()

    @pl.when(e + 1 < E)
    def _():
        for c in range(NC):
            pltpu.make_async_copy(
                w1_hbm.at[e + 1, pl.ds(c * DC, DC), :],
                w1_buf.at[nslot, pl.ds(c * DC, DC), :],
                sems.at[nslot, c]).start()

    for c in range(NC):
        pltpu.make_async_copy(
            w1_hbm.at[e, pl.ds(c * DC, DC), :],
            w1_buf.at[slot, pl.ds(c * DC, DC), :],
            sems.at[slot, c]).wait()

    start = meta_s[NW + e]
    cnt = meta_s[NW + E + e]
    nb = (cnt + CB - 1) // CB

    def _perm(p):
        word = meta_s[p // 4]
        return (word >> (8 * (p % 4))) & 0xFF

    def blk(k, carry):
        base = k * CB
        for i in range(CB):
            p = jnp.minimum(start + base + i, B - 1)
            b = _perm(p)
            xs_ref[pl.ds(i * T, T), :] = x_ref[pl.ds(b * T, T), :]
        h = jnp.dot(xs_ref[...], w1_buf[slot],
                    preferred_element_type=jnp.float32) + b1_ref[0]
        h = 0.5 * h * (1.0 + jax.lax.erf(h * _INV_SQRT2))
        dec = jnp.dot(h, w2_ref[0], preferred_element_type=jnp.float32)
        dec = (dec + b2_ref[0]) * mask_ref[0]
        for i in range(CB):
            p = jnp.minimum(start + base + i, B - 1)
            b = _perm(p)

            @pl.when(base + i < cnt)
            def _store():
                out_ref[pl.ds(b * T, T), :] = dec[i * T:(i + 1) * T, :]

        return carry

    jax.lax.fori_loop(0, nb, blk, 0)


def kernel(pred_action_latents, embodiment_ids, W1, b1, W2, b2, action_mask):
    Bn, Tn, _ = pred_action_latents.shape
    N = Bn * Tn
    x = pred_action_latents.reshape(N, D)

    ids = embodiment_ids.astype(jnp.int32)
    order = jnp.argsort(ids, stable=True).astype(jnp.int32)
    counts = jnp.zeros((E,), jnp.int32).at[ids].add(1)
    starts = jnp.concatenate(
        [jnp.zeros((1,), jnp.int32), jnp.cumsum(counts)[:-1].astype(jnp.int32)])
    ow = order.reshape(NW, 4)
    packed = (ow[:, 0] | (ow[:, 1] << 8) | (ow[:, 2] << 16) | (ow[:, 3] << 24))
    meta = jnp.concatenate([packed, starts, counts])

    out = pl.pallas_call(
        _mlp_kernel,
        grid=(E,),
        in_specs=[
            pl.BlockSpec(memory_space=pl.ANY),
            pl.BlockSpec((N, D), lambda e: (0, 0)),
            pl.BlockSpec(memory_space=pl.ANY),
            pl.BlockSpec((1, 1, H_DIM), lambda e: (e, 0, 0)),
            pl.BlockSpec((1, H_DIM, MAX_A), lambda e: (e, 0, 0)),
            pl.BlockSpec((1, 1, MAX_A), lambda e: (e, 0, 0)),
            pl.BlockSpec((1, 1, MAX_A), lambda e: (e, 0, 0)),
        ],
        out_specs=pl.BlockSpec((N, MAX_A), lambda e: (0, 0)),
        scratch_shapes=[
            pltpu.VMEM((CB * T, D), jnp.float32),
            pltpu.VMEM((2, D, H_DIM), jnp.float32),
            pltpu.SMEM((META_LEN,), jnp.int32),
            pltpu.SemaphoreType.DMA((2, NC)),
            pltpu.SemaphoreType.DMA,
        ],
        out_shape=jax.ShapeDtypeStruct((N, MAX_A), jnp.float32),
    )(meta, x, W1, b1[:, None, :], W2, b2[:, None, :],
      action_mask[:, None, :])
    return out.reshape(Bn, Tn, MAX_A)


# speedup vs baseline: 1.9432x; 1.5284x over previous
import jax
import jax.numpy as jnp
from jax.experimental import pallas as pl
from jax.experimental.pallas import tpu as pltpu

E = 8
D = 1024
H_DIM = 2048
MAX_A = 32
T = 8
B = 128
CB = 16
NC = 32
DC = D // NC

_INV_SQRT2 = 0.7071067811865476


def _mlp_kernel(ids_hbm, x_ref, w1_hbm, b1_ref, w2_ref, b2_ref, mask_ref,
                out_ref, xs_ref, w1_buf, ids_s, perm_s, cnt_s, start_s,
                offs_s, sems, msem):
    e = pl.program_id(0)
    slot = jax.lax.rem(e, 2)
    nslot = jax.lax.rem(e + 1, 2)

    @pl.when(e == 0)
    def _():
        pltpu.make_async_copy(ids_hbm, ids_s, msem).start()
        for c in range(NC):
            pltpu.make_async_copy(
                w1_hbm.at[0, pl.ds(c * DC, DC), :],
                w1_buf.at[0, pl.ds(c * DC, DC), :],
                sems.at[0, c]).start()
        pltpu.make_async_copy(ids_hbm, ids_s, msem).wait()
        for j in range(E):
            cnt_s[j] = 0

        def _count(p, carry):
            idp = ids_s[p]
            cnt_s[idp] = cnt_s[idp] + 1
            return carry

        jax.lax.fori_loop(0, B, _count, 0)
        s = 0
        for j in range(E):
            start_s[j] = s
            offs_s[j] = s
            s = s + cnt_s[j]

        def _place(p, carry):
            idp = ids_s[p]
            o = offs_s[idp]
            perm_s[o] = p
            offs_s[idp] = o + 1
            return carry

        jax.lax.fori_loop(0, B, _place, 0)

    @pl.when(e + 1 < E)
    def _():
        for c in range(NC):
            pltpu.make_async_copy(
                w1_hbm.at[e + 1, pl.ds(c * DC, DC), :],
                w1_buf.at[nslot, pl.ds(c * DC, DC), :],
                sems.at[nslot, c]).start()

    for c in range(NC):
        pltpu.make_async_copy(
            w1_hbm.at[e, pl.ds(c * DC, DC), :],
            w1_buf.at[slot, pl.ds(c * DC, DC), :],
            sems.at[slot, c]).wait()

    start = start_s[e]
    cnt = cnt_s[e]
    nb = (cnt + CB - 1) // CB

    def blk(k, carry):
        base = k * CB
        for i in range(CB):
            p = jnp.minimum(start + base + i, B - 1)
            b = perm_s[p]
            xs_ref[pl.ds(i * T, T), :] = x_ref[pl.ds(b * T, T), :]
        h = jnp.dot(xs_ref[...], w1_buf[slot],
                    preferred_element_type=jnp.float32) + b1_ref[0]
        h = 0.5 * h * (1.0 + jax.lax.erf(h * _INV_SQRT2))
        dec = jnp.dot(h, w2_ref[0], preferred_element_type=jnp.float32)
        dec = (dec + b2_ref[0]) * mask_ref[0]
        for i in range(CB):
            p = jnp.minimum(start + base + i, B - 1)
            b = perm_s[p]

            @pl.when(base + i < cnt)
            def _store():
                out_ref[pl.ds(b * T, T), :] = dec[i * T:(i + 1) * T, :]

        return carry

    jax.lax.fori_loop(0, nb, blk, 0)


def kernel(pred_action_latents, embodiment_ids, W1, b1, W2, b2, action_mask):
    Bn, Tn, _ = pred_action_latents.shape
    N = Bn * Tn
    x = pred_action_latents.reshape(N, D)

    out = pl.pallas_call(
        _mlp_kernel,
        grid=(E,),
        in_specs=[
            pl.BlockSpec(memory_space=pl.ANY),
            pl.BlockSpec((N, D), lambda e: (0, 0)),
            pl.BlockSpec(memory_space=pl.ANY),
            pl.BlockSpec((1, 1, H_DIM), lambda e: (e, 0, 0)),
            pl.BlockSpec((1, H_DIM, MAX_A), lambda e: (e, 0, 0)),
            pl.BlockSpec((1, 1, MAX_A), lambda e: (e, 0, 0)),
            pl.BlockSpec((1, 1, MAX_A), lambda e: (e, 0, 0)),
        ],
        out_specs=pl.BlockSpec((N, MAX_A), lambda e: (0, 0)),
        scratch_shapes=[
            pltpu.VMEM((CB * T, D), jnp.float32),
            pltpu.VMEM((2, D, H_DIM), jnp.float32),
            pltpu.SMEM((B,), jnp.int32),
            pltpu.SMEM((B,), jnp.int32),
            pltpu.SMEM((E,), jnp.int32),
            pltpu.SMEM((E,), jnp.int32),
            pltpu.SMEM((E,), jnp.int32),
            pltpu.SemaphoreType.DMA((2, NC)),
            pltpu.SemaphoreType.DMA,
        ],
        out_shape=jax.ShapeDtypeStruct((N, MAX_A), jnp.float32),
    )(embodiment_ids.astype(jnp.int32), x, W1, b1[:, None, :], W2,
      b2[:, None, :], action_mask[:, None, :])
    return out.reshape(Bn, Tn, MAX_A)


# speedup vs baseline: 1.9763x; 1.0171x over previous
import jax
import jax.numpy as jnp
from jax.experimental import pallas as pl
from jax.experimental.pallas import tpu as pltpu

E = 8
D = 1024
H_DIM = 2048
MAX_A = 32
T = 8
B = 128
CB = 16
NC = 32
DC = D // NC

_INV_SQRT2 = 0.7071067811865476


def _mlp_kernel(ids_hbm, x_ref, w1_hbm, b1_hbm, w2_hbm, b2_hbm, mask_hbm,
                out_ref, xs_ref, w1_buf, w2_s, b1_s, b2_s, mask_s,
                ids_s, perm_s, cnt_s, start_s, offs_s, sems, ssems):
    e = pl.program_id(0)
    slot = jax.lax.rem(e, 2)
    nslot = jax.lax.rem(e + 1, 2)

    @pl.when(e == 0)
    def _():
        pltpu.make_async_copy(ids_hbm, ids_s, ssems.at[0]).start()
        pltpu.make_async_copy(w2_hbm, w2_s, ssems.at[1]).start()
        pltpu.make_async_copy(b1_hbm, b1_s, ssems.at[2]).start()
        pltpu.make_async_copy(b2_hbm, b2_s, ssems.at[3]).start()
        pltpu.make_async_copy(mask_hbm, mask_s, ssems.at[4]).start()
        for c in range(NC):
            pltpu.make_async_copy(
                w1_hbm.at[0, pl.ds(c * DC, DC), :],
                w1_buf.at[0, pl.ds(c * DC, DC), :],
                sems.at[0, c]).start()
        pltpu.make_async_copy(ids_hbm, ids_s, ssems.at[0]).wait()
        for j in range(E):
            cnt_s[j] = 0

        def _count(p, carry):
            idp = ids_s[p]
            cnt_s[idp] = cnt_s[idp] + 1
            return carry

        jax.lax.fori_loop(0, B, _count, 0)
        s = 0
        for j in range(E):
            start_s[j] = s
            offs_s[j] = s
            s = s + cnt_s[j]

        def _place(p, carry):
            idp = ids_s[p]
            o = offs_s[idp]
            perm_s[o] = p
            offs_s[idp] = o + 1
            return carry

        jax.lax.fori_loop(0, B, _place, 0)
        pltpu.make_async_copy(w2_hbm, w2_s, ssems.at[1]).wait()
        pltpu.make_async_copy(b1_hbm, b1_s, ssems.at[2]).wait()
        pltpu.make_async_copy(b2_hbm, b2_s, ssems.at[3]).wait()
        pltpu.make_async_copy(mask_hbm, mask_s, ssems.at[4]).wait()

    @pl.when(e + 1 < E)
    def _():
        for c in range(NC):
            pltpu.make_async_copy(
                w1_hbm.at[e + 1, pl.ds(c * DC, DC), :],
                w1_buf.at[nslot, pl.ds(c * DC, DC), :],
                sems.at[nslot, c]).start()

    for c in range(NC):
        pltpu.make_async_copy(
            w1_hbm.at[e, pl.ds(c * DC, DC), :],
            w1_buf.at[slot, pl.ds(c * DC, DC), :],
            sems.at[slot, c]).wait()

    start = start_s[e]
    cnt = cnt_s[e]
    nb = (cnt + CB - 1) // CB

    def blk(k, carry):
        base = k * CB
        for i in range(CB):
            p = jnp.minimum(start + base + i, B - 1)
            b = perm_s[p]
            xs_ref[pl.ds(i * T, T), :] = x_ref[pl.ds(b * T, T), :]
        h = jnp.dot(xs_ref[...], w1_buf[slot],
                    preferred_element_type=jnp.float32) + b1_s[e]
        h = 0.5 * h * (1.0 + jax.lax.erf(h * _INV_SQRT2))
        dec = jnp.dot(h, w2_s[e], preferred_element_type=jnp.float32)
        dec = (dec + b2_s[e]) * mask_s[e]
        for i in range(CB):
            p = jnp.minimum(start + base + i, B - 1)
            b = perm_s[p]

            @pl.when(base + i < cnt)
            def _store():
                out_ref[pl.ds(b * T, T), :] = dec[i * T:(i + 1) * T, :]

        return carry

    jax.lax.fori_loop(0, nb, blk, 0)


def kernel(pred_action_latents, embodiment_ids, W1, b1, W2, b2, action_mask):
    Bn, Tn, _ = pred_action_latents.shape
    N = Bn * Tn
    x = pred_action_latents.reshape(N, D)

    out = pl.pallas_call(
        _mlp_kernel,
        grid=(E,),
        in_specs=[
            pl.BlockSpec(memory_space=pl.ANY),
            pl.BlockSpec((N, D), lambda e: (0, 0)),
            pl.BlockSpec(memory_space=pl.ANY),
            pl.BlockSpec(memory_space=pl.ANY),
            pl.BlockSpec(memory_space=pl.ANY),
            pl.BlockSpec(memory_space=pl.ANY),
            pl.BlockSpec(memory_space=pl.ANY),
        ],
        out_specs=pl.BlockSpec((N, MAX_A), lambda e: (0, 0)),
        scratch_shapes=[
            pltpu.VMEM((CB * T, D), jnp.float32),
            pltpu.VMEM((2, D, H_DIM), jnp.float32),
            pltpu.VMEM((E, H_DIM, MAX_A), jnp.float32),
            pltpu.VMEM((E, H_DIM), jnp.float32),
            pltpu.VMEM((E, MAX_A), jnp.float32),
            pltpu.VMEM((E, MAX_A), jnp.float32),
            pltpu.SMEM((B,), jnp.int32),
            pltpu.SMEM((B,), jnp.int32),
            pltpu.SMEM((E,), jnp.int32),
            pltpu.SMEM((E,), jnp.int32),
            pltpu.SMEM((E,), jnp.int32),
            pltpu.SemaphoreType.DMA((2, NC)),
            pltpu.SemaphoreType.DMA((5,)),
        ],
        out_shape=jax.ShapeDtypeStruct((N, MAX_A), jnp.float32),
    )(embodiment_ids.astype(jnp.int32), x, W1, b1, W2, b2, action_mask)
    return out.reshape(Bn, Tn, MAX_A)
